# baseline (speedup 1.0000x reference)
import functools

import jax
import jax.numpy as jnp
from jax import lax
from jax.experimental import pallas as pl
from jax.experimental.pallas import tpu as pltpu
from jax.experimental.pallas import tpu_sc as plsc

B = 4
S = 4096
D = 2048
E = 64
K = 8
SBLK = 1024
NBLK = S // SBLK


def _tc_body(vt_ref, text_ref, w1_ref, w2_ref, b_ref,
             probs_ref, ent_ref, aux_ref,
             acc_ref, hrows_ref):
    bidx = pl.program_id(0)
    sidx = pl.program_id(1)

    @pl.when(sidx == 0)
    def _init():
        acc_ref[...] = jnp.zeros_like(acc_ref)

    acc_ref[...] += jnp.sum(vt_ref[0].reshape(SBLK // 8, 8, D), axis=0)

    for k in range(B):
        @pl.when((bidx == k) & (sidx == NBLK - 1))
        def _stash():
            hrows_ref[8 * k:8 * k + 8, :] = acc_ref[...]

    @pl.when((bidx == B - 1) & (sidx == NBLK - 1))
    def _finish():
        h = jnp.sum(hrows_ref[...].reshape(B, 8, D), axis=1) * (1.0 / S)
        logits = (jnp.dot(h, w1_ref[...], preferred_element_type=jnp.float32)
                  + jnp.dot(text_ref[...], w2_ref[...],
                            preferred_element_type=jnp.float32)
                  + b_ref[...])
        m = jnp.max(logits, axis=-1, keepdims=True)
        ex = jnp.exp(logits - m)
        probs = ex / jnp.sum(ex, axis=-1, keepdims=True)
        probs_ref[...] = probs

        ent = -jnp.sum(probs * jnp.log(probs + 1e-8)) * (1.0 / B)
        ent_ref[...] = ent.reshape(1, 1)
        mu = jnp.mean(probs, axis=0, keepdims=True)
        aux_ref[...] = jnp.mean((probs - mu) ** 2).reshape(1, 1)


def _tc_mean_head(video_tokens, text_state, w1, w2, b2):
    return pl.pallas_call(
        _tc_body,
        grid=(B, NBLK),
        in_specs=[
            pl.BlockSpec((1, SBLK, D), lambda bi, si: (bi, si, 0)),
            pl.BlockSpec((B, D), lambda bi, si: (0, 0)),
            pl.BlockSpec((D, E), lambda bi, si: (0, 0)),
            pl.BlockSpec((D, E), lambda bi, si: (0, 0)),
            pl.BlockSpec((1, E), lambda bi, si: (0, 0)),
        ],
        out_specs=[
            pl.BlockSpec((B, E), lambda bi, si: (0, 0)),
            pl.BlockSpec((1, 1), lambda bi, si: (0, 0)),
            pl.BlockSpec((1, 1), lambda bi, si: (0, 0)),
        ],
        out_shape=[
            jax.ShapeDtypeStruct((B, E), jnp.float32),
            jax.ShapeDtypeStruct((1, 1), jnp.float32),
            jax.ShapeDtypeStruct((1, 1), jnp.float32),
        ],
        scratch_shapes=[pltpu.VMEM((8, D), jnp.float32),
                        pltpu.VMEM((8 * B, D), jnp.float32)],
    )(video_tokens, text_state, w1, w2, b2)


def _sc_topk_body(probs_hbm, topv_hbm, topi_hbm, row_v, tv_v, ti_v):
    wid = lax.axis_index("c") * 16 + lax.axis_index("s")

    @pl.when(wid < B)
    def _():
        pltpu.sync_copy(probs_hbm.at[wid], row_v)

        lane = lax.iota(jnp.int32, 16)
        neg = jnp.full((16,), -jnp.inf, jnp.float32)
        perms = [lane ^ s for s in (1, 2, 4, 8)]
        chunks = [row_v[pl.ds(i * 16, 16)] for i in range(E // 16)]
        tv = jnp.zeros((16,), jnp.float32)
        ti = jnp.zeros((16,), jnp.int32)
        ssum = jnp.zeros((16,), jnp.float32)
        for k in range(K):
            bv = chunks[0]
            bi = lane
            for j in range(1, E // 16):
                cj = chunks[j]
                take = cj > bv
                bi = jnp.where(take, lane + 16 * j, bi)
                bv = jnp.where(take, cj, bv)
            for p in perms:
                pv = bv.at[p].get(mode="promise_in_bounds")
                pi = bi.at[p].get(mode="promise_in_bounds")
                take = (pv > bv) | ((pv == bv) & (pi < bi))
                bv = jnp.where(take, pv, bv)
                bi = jnp.where(take, pi, bi)
            tv = jnp.where(lane == k, bv, tv)
            ti = jnp.where(lane == k, bi, ti)
            ssum = ssum + bv
            chunks = [jnp.where(lane + 16 * j == bi, neg, chunks[j])
                      for j in range(E // 16)]
        tv_v[...] = tv / (ssum + 1e-8)
        ti_v[...] = ti
        pltpu.sync_copy(tv_v, topv_hbm.at[wid])
        pltpu.sync_copy(ti_v, topi_hbm.at[wid])


_sc_topk = pl.kernel(
    _sc_topk_body,
    out_type=[
        jax.ShapeDtypeStruct((B, 16), jnp.float32),
        jax.ShapeDtypeStruct((B, 16), jnp.int32),
    ],
    mesh=plsc.VectorSubcoreMesh(core_axis_name="c", subcore_axis_name="s"),
    scratch_types=[
        pltpu.VMEM((E,), jnp.float32),
        pltpu.VMEM((16,), jnp.float32),
        pltpu.VMEM((16,), jnp.int32),
    ],
)


@functools.partial(jax.jit, static_argnames=())
def kernel(video_tokens, text_state, W, b):
    w1 = W[:D]
    w2 = W[D:]
    b2 = b.reshape(1, E)
    probs, ent, aux = _tc_mean_head(video_tokens, text_state, w1, w2, b2)
    topv16, topi16 = _sc_topk(probs)
    return (topi16[:, :K], topv16[:, :K], probs,
            ent.reshape(()), aux.reshape(()))
